# baseline (speedup 1.0000x reference)
import dataclasses

import jax
import jax.numpy as jnp
from jax import lax
from jax.experimental import pallas as pl
from jax.experimental.pallas import tpu as pltpu
from jax.experimental.pallas import tpu_sc as plsc

N_NODES = 10000
N_EDGES = 320000
D = 128

NC = 2
NS = 16
NW = NC * NS
LANES = 16

N_PAD = 10240
ROWS_W = N_PAD // NS
SENT = N_PAD - 1
EPW = N_PAD
CW = 80
RPB = 8
BLKE = RPB * CW
NBLK = EPW // BLKE
NEG_BIG = -3.4e38
SENT_VAL = -1e30


def _sc_compiler_params():
    cp = pltpu.CompilerParams()
    if "needs_layout_passes" in pltpu.CompilerParams.__dataclass_fields__:
        cp = dataclasses.replace(cp, needs_layout_passes=False)
    return cp


def _proj_body(h_ref, w_ref, o_ref):
    o_ref[...] = jnp.dot(h_ref[...], w_ref[...],
                         preferred_element_type=jnp.float32)


def _project(h, w2):
    n = h.shape[0]
    blk = 1000
    return pl.pallas_call(
        _proj_body,
        grid=(n // blk,),
        in_specs=[
            pl.BlockSpec((blk, D), lambda i: (i, 0)),
            pl.BlockSpec((D, 256), lambda i: (0, 0)),
        ],
        out_specs=pl.BlockSpec((blk, 256), lambda i: (i, 0)),
        out_shape=jax.ShapeDtypeStruct((n, 256), jnp.float32),
    )(h, w2)


def _sca_body(s_hbm, d_hbm, src_hbm, dst_hbm,
              ex_hbm, den_hbm, mx_hbm,
              s_v, d_v, e_v, denom_v, srcblk_v, dstblk_v, e_blk, m_v, stage_v,
              stage_sh):
    cid = lax.axis_index("c")
    sid = lax.axis_index("s")
    wid = cid * NS + sid

    pltpu.sync_copy(s_hbm, s_v)
    pltpu.sync_copy(d_hbm, d_v)
    m_v[...] = jnp.full((LANES,), NEG_BIG, jnp.float32)

    @pl.loop(0, N_PAD, step=LANES)
    def _(i):
        denom_v[pl.ds(i, LANES)] = jnp.zeros((LANES,), jnp.float32)

    @pl.loop(0, NBLK)
    def _(b):
        pltpu.sync_copy(src_hbm.at[wid, pl.ds(b * RPB, RPB)], srcblk_v)
        pltpu.sync_copy(dst_hbm.at[wid, pl.ds(b * RPB, RPB)], dstblk_v)

        @pl.loop(0, RPB)
        def _(r):
            @pl.loop(0, CW, step=LANES)
            def _(k):
                s16 = plsc.load_gather(s_v, [srcblk_v.at[r][pl.ds(k, LANES)]])
                d16 = plsc.load_gather(d_v, [dstblk_v.at[r][pl.ds(k, LANES)]])
                x = s16 + d16
                e16 = jnp.where(x >= 0.0, x, x * jnp.float32(0.01))
                e_v[pl.ds(b * BLKE + r * CW + k, LANES)] = e16
                m_v[...] = jnp.maximum(m_v[...], e16)

    pltpu.sync_copy(m_v, stage_sh.at[sid])
    plsc.subcore_barrier()
    pltpu.sync_copy(stage_sh, stage_v)

    @pl.loop(0, NS)
    def _(i):
        m_v[...] = jnp.maximum(m_v[...], stage_v.at[i][...])

    @pl.when(sid == 0)
    def _():
        pltpu.sync_copy(m_v, mx_hbm.at[cid, 0])

    msc = jnp.max(m_v[...])

    @pl.loop(0, NBLK)
    def _(b):
        pltpu.sync_copy(dst_hbm.at[wid, pl.ds(b * RPB, RPB)], dstblk_v)

        @pl.loop(0, RPB)
        def _(r):
            @pl.loop(0, CW, step=LANES)
            def _(k):
                ex = jnp.exp(e_v[pl.ds(b * BLKE + r * CW + k, LANES)] - msc)
                e_blk[pl.ds(r * CW + k, LANES)] = ex
                plsc.addupdate_scatter(
                    denom_v, [dstblk_v.at[r][pl.ds(k, LANES)]], ex)

        pltpu.sync_copy(e_blk, ex_hbm.at[pl.ds(wid * EPW + b * BLKE, BLKE)])

    pltpu.sync_copy(denom_v, den_hbm.at[pl.ds(wid * N_PAD, N_PAD)])


def _sc_logits(s_pad, d_pad, src3, dst3):
    mesh = plsc.VectorSubcoreMesh(core_axis_name="c", subcore_axis_name="s")
    kern = pl.kernel(
        _sca_body,
        out_type=[
            jax.ShapeDtypeStruct((NW * EPW,), jnp.float32),
            jax.ShapeDtypeStruct((NW * N_PAD,), jnp.float32),
            jax.ShapeDtypeStruct((NC, 8, LANES), jnp.float32),
        ],
        mesh=mesh,
        scratch_types=[
            pltpu.VMEM((N_PAD,), jnp.float32),
            pltpu.VMEM((N_PAD,), jnp.float32),
            pltpu.VMEM((EPW,), jnp.float32),
            pltpu.VMEM((N_PAD,), jnp.float32),
            pltpu.VMEM((RPB, CW), jnp.int32),
            pltpu.VMEM((RPB, CW), jnp.int32),
            pltpu.VMEM((BLKE,), jnp.float32),
            pltpu.VMEM((LANES,), jnp.float32),
            pltpu.VMEM((NS, LANES), jnp.float32),
            pltpu.VMEM_SHARED((NS, LANES), jnp.float32),
        ],
        compiler_params=_sc_compiler_params(),
    )
    return kern(s_pad, d_pad, src3, dst3)


def _scb_body(z_hbm, src_hbm, dst_hbm, ex_hbm, zeros_hbm,
              p_hbm,
              srcblk_v, dstblk_v, eexp_blk,
              z0, z1, z2, z3,
              p_sh,
              sg0, sg1, sg2, sg3, ss0, ss1, ss2, ss3):
    cid = lax.axis_index("c")
    sid = lax.axis_index("s")
    wid = cid * NS + sid
    cbase = sid * ROWS_W

    pltpu.sync_copy(zeros_hbm, p_sh.at[pl.ds(cbase, ROWS_W)])
    plsc.subcore_barrier()

    zbufs = (z0, z1, z2, z3)
    gsems = (sg0, sg1, sg2, sg3)
    ssems = (ss0, ss1, ss2, ss3)
    NBUF = 4

    def scale_chunk(buf, r):
        @plsc.parallel_loop(0, CW, unroll=4)
        def _(j):
            bc = plsc.load_gather(
                eexp_blk, [jnp.zeros((LANES,), jnp.int32) + (r * CW + j)])
            row = buf.at[j]
            for t in range(D // LANES):
                row[pl.ds(t * LANES, LANES)] = (
                    row[pl.ds(t * LANES, LANES)] * bc)

    @pl.loop(0, NBLK)
    def _(b):
        pltpu.sync_copy(src_hbm.at[wid, pl.ds(b * RPB, RPB)], srcblk_v)
        pltpu.sync_copy(dst_hbm.at[wid, pl.ds(b * RPB, RPB)], dstblk_v)
        pltpu.sync_copy(ex_hbm.at[pl.ds(wid * EPW + b * BLKE, BLKE)], eexp_blk)

        gds = [None] * RPB
        sds = [None] * RPB
        for r in range(NBUF - 1):
            gds[r] = pltpu.async_copy(
                z_hbm.at[srcblk_v.at[r]], zbufs[r % NBUF], gsems[r % NBUF])
        for r in range(RPB):
            buf = zbufs[r % NBUF]
            gds[r].wait()
            scale_chunk(buf, r)
            sds[r] = pltpu.async_copy(
                buf, p_sh.at[dstblk_v.at[r]], ssems[r % NBUF], add=True)
            if r + NBUF - 1 < RPB:
                nb = (r + NBUF - 1) % NBUF
                if r >= 1 and sds[r - 1] is not None:
                    sds[r - 1].wait()
                gds[r + NBUF - 1] = pltpu.async_copy(
                    z_hbm.at[srcblk_v.at[r + NBUF - 1]], zbufs[nb], gsems[nb])
        for r in range(RPB - NBUF, RPB):
            if r >= 0 and sds[r] is not None:
                sds[r].wait()

    plsc.subcore_barrier()

    pltpu.sync_copy(p_sh.at[pl.ds(cbase, ROWS_W)],
                    p_hbm.at[cid, pl.ds(cbase, ROWS_W)])


def _sc_aggregate(z_pad, src3, dst3, ex_all, zeros_rows):
    mesh = plsc.VectorSubcoreMesh(core_axis_name="c", subcore_axis_name="s")
    kern = pl.kernel(
        _scb_body,
        out_type=[
            jax.ShapeDtypeStruct((NC, N_PAD, D), jnp.float32),
        ],
        mesh=mesh,
        scratch_types=[
            pltpu.VMEM((RPB, CW), jnp.int32),
            pltpu.VMEM((RPB, CW), jnp.int32),
            pltpu.VMEM((BLKE,), jnp.float32),
            pltpu.VMEM((CW, D), jnp.float32),
            pltpu.VMEM((CW, D), jnp.float32),
            pltpu.VMEM((CW, D), jnp.float32),
            pltpu.VMEM((CW, D), jnp.float32),
            pltpu.VMEM_SHARED((N_PAD, D), jnp.float32),
            pltpu.SemaphoreType.DMA,
            pltpu.SemaphoreType.DMA,
            pltpu.SemaphoreType.DMA,
            pltpu.SemaphoreType.DMA,
            pltpu.SemaphoreType.DMA,
            pltpu.SemaphoreType.DMA,
            pltpu.SemaphoreType.DMA,
            pltpu.SemaphoreType.DMA,
        ],
        compiler_params=_sc_compiler_params(),
    )
    return kern(z_pad, src3, dst3, ex_all, zeros_rows)


def _combine_body(p0_ref, p1_ref, d_ref, c_ref, o_ref):
    c0 = c_ref[0]
    c1 = c_ref[1]
    den = (jnp.sum(d_ref[0], axis=0) * c0
           + jnp.sum(d_ref[1], axis=0) * c1)[:, None]
    den = jnp.where(den == 0.0, jnp.float32(1.0), den)
    o_ref[...] = (p0_ref[...] * c0 + p1_ref[...] * c1) / den


def _combine(p0, p1, dall, c):
    blk = 1024
    return pl.pallas_call(
        _combine_body,
        grid=(N_PAD // blk,),
        in_specs=[
            pl.BlockSpec((blk, D), lambda i: (i, 0)),
            pl.BlockSpec((blk, D), lambda i: (i, 0)),
            pl.BlockSpec((NC, NS, blk), lambda i: (0, 0, i)),
            pl.BlockSpec(memory_space=pltpu.SMEM),
        ],
        out_specs=pl.BlockSpec((blk, D), lambda i: (i, 0)),
        out_shape=jax.ShapeDtypeStruct((N_PAD, D), jnp.float32),
    )(p0, p1, dall, c)


@jax.jit
def kernel(h, edge_index, W_fc, W_attn):
    a_src = W_attn[0, :D]
    a_dst = W_attn[0, D:]
    wt = W_fc.T
    v1 = wt @ a_src
    v2 = wt @ a_dst
    w2 = jnp.zeros((D, 256), jnp.float32)
    w2 = w2.at[:, :D].set(wt).at[:, D].set(v1).at[:, D + 1].set(v2)

    z2 = _project(h, w2)
    z_pad = jnp.zeros((N_PAD, D), jnp.float32).at[:N_NODES].set(z2[:, :D])
    s_pad = (jnp.zeros((N_PAD,), jnp.float32).at[:N_NODES].set(z2[:, D])
             .at[SENT].set(jnp.float32(SENT_VAL)))
    d_pad = (jnp.zeros((N_PAD,), jnp.float32).at[:N_NODES].set(z2[:, D + 1])
             .at[SENT].set(jnp.float32(SENT_VAL)))

    def slab(row):
        a = row.astype(jnp.int32).reshape(NW, N_EDGES // NW)
        a = jnp.pad(a, ((0, 0), (0, EPW - N_EDGES // NW)),
                    constant_values=SENT)
        return a.reshape(NW, EPW // CW, CW)

    src3 = slab(edge_index[0])
    dst3 = slab(edge_index[1])
    zeros_rows = jnp.zeros((ROWS_W, D), jnp.float32)

    ex_all, den, mx = _sc_logits(s_pad, d_pad, src3, dst3)
    (p,) = _sc_aggregate(z_pad, src3, dst3, ex_all, zeros_rows)
    dall = den.reshape(NC, NS, N_PAD)

    m0 = jnp.max(mx[0, 0])
    m1 = jnp.max(mx[1, 0])
    mg = jnp.maximum(m0, m1)
    c = jnp.stack([jnp.exp(m0 - mg), jnp.exp(m1 - mg)])

    sh = _combine(p[0], p[1], dall, c)
    return sh[:N_NODES]


# speedup vs baseline: 1.0219x; 1.0219x over previous
import dataclasses

import jax
import jax.numpy as jnp
from jax import lax
from jax.experimental import pallas as pl
from jax.experimental.pallas import tpu as pltpu
from jax.experimental.pallas import tpu_sc as plsc

N_NODES = 10000
N_EDGES = 320000
D = 128

NC = 2
NS = 16
NW = NC * NS
LANES = 16

N_PAD = 10240
ROWS_W = N_PAD // NS
SENT = N_PAD - 1
EPW = N_PAD
CW = 80
RPB = 8
BLKE = RPB * CW
NBLK = EPW // BLKE
NEG_BIG = -3.4e38
SENT_VAL = -1e30


def _sc_compiler_params():
    cp = pltpu.CompilerParams()
    if "needs_layout_passes" in pltpu.CompilerParams.__dataclass_fields__:
        cp = dataclasses.replace(cp, needs_layout_passes=False)
    return cp


def _proj_body(h_ref, w_ref, o_ref):
    o_ref[...] = jnp.dot(h_ref[...], w_ref[...],
                         preferred_element_type=jnp.float32)


def _project(h, w2):
    n = h.shape[0]
    blk = 1000
    return pl.pallas_call(
        _proj_body,
        grid=(n // blk,),
        in_specs=[
            pl.BlockSpec((blk, D), lambda i: (i, 0)),
            pl.BlockSpec((D, 256), lambda i: (0, 0)),
        ],
        out_specs=pl.BlockSpec((blk, 256), lambda i: (i, 0)),
        out_shape=jax.ShapeDtypeStruct((n, 256), jnp.float32),
    )(h, w2)


def _sca_body(s_hbm, d_hbm, src_hbm, dst_hbm,
              ex_hbm, den_hbm, mx_hbm,
              s_v, d_v, e_v, denom_v, srcblk_v, dstblk_v, e_blk, m_v, stage_v,
              stage_sh):
    cid = lax.axis_index("c")
    sid = lax.axis_index("s")
    wid = cid * NS + sid

    pltpu.sync_copy(s_hbm, s_v)
    pltpu.sync_copy(d_hbm, d_v)
    @pl.loop(0, N_PAD, step=LANES)
    def _(i):
        denom_v[pl.ds(i, LANES)] = jnp.zeros((LANES,), jnp.float32)

    @pl.loop(0, NBLK, init_carry=jnp.full((LANES,), NEG_BIG, jnp.float32))
    def mfin(b, m_blk):
        pltpu.sync_copy(src_hbm.at[wid, pl.ds(b * RPB, RPB)], srcblk_v)
        pltpu.sync_copy(dst_hbm.at[wid, pl.ds(b * RPB, RPB)], dstblk_v)

        @pl.loop(0, RPB, init_carry=m_blk)
        def m_r(r, m_row):
            @plsc.parallel_loop(0, CW, LANES, unroll=4, carry=m_row)
            def m_k(k, m_acc):
                s16 = plsc.load_gather(s_v, [srcblk_v.at[r][pl.ds(k, LANES)]])
                d16 = plsc.load_gather(d_v, [dstblk_v.at[r][pl.ds(k, LANES)]])
                x = s16 + d16
                e16 = jnp.where(x >= 0.0, x, x * jnp.float32(0.01))
                e_v[pl.ds(b * BLKE + r * CW + k, LANES)] = e16
                return jnp.maximum(m_acc, e16)
            return m_k
        return m_r

    m_v[...] = mfin

    pltpu.sync_copy(m_v, stage_sh.at[sid])
    plsc.subcore_barrier()
    pltpu.sync_copy(stage_sh, stage_v)

    @pl.loop(0, NS)
    def _(i):
        m_v[...] = jnp.maximum(m_v[...], stage_v.at[i][...])

    @pl.when(sid == 0)
    def _():
        pltpu.sync_copy(m_v, mx_hbm.at[cid, 0])

    msc = jnp.max(m_v[...])

    @pl.loop(0, NBLK)
    def _(b):
        pltpu.sync_copy(dst_hbm.at[wid, pl.ds(b * RPB, RPB)], dstblk_v)

        @pl.loop(0, RPB)
        def _(r):
            @plsc.parallel_loop(0, CW, LANES, unroll=4)
            def _(k):
                ex = jnp.exp(e_v[pl.ds(b * BLKE + r * CW + k, LANES)] - msc)
                e_blk[pl.ds(r * CW + k, LANES)] = ex
                plsc.addupdate_scatter(
                    denom_v, [dstblk_v.at[r][pl.ds(k, LANES)]], ex)

        pltpu.sync_copy(e_blk, ex_hbm.at[pl.ds(wid * EPW + b * BLKE, BLKE)])

    pltpu.sync_copy(denom_v, den_hbm.at[pl.ds(wid * N_PAD, N_PAD)])


def _sc_logits(s_pad, d_pad, src3, dst3):
    mesh = plsc.VectorSubcoreMesh(core_axis_name="c", subcore_axis_name="s")
    kern = pl.kernel(
        _sca_body,
        out_type=[
            jax.ShapeDtypeStruct((NW * EPW,), jnp.float32),
            jax.ShapeDtypeStruct((NW * N_PAD,), jnp.float32),
            jax.ShapeDtypeStruct((NC, 8, LANES), jnp.float32),
        ],
        mesh=mesh,
        scratch_types=[
            pltpu.VMEM((N_PAD,), jnp.float32),
            pltpu.VMEM((N_PAD,), jnp.float32),
            pltpu.VMEM((EPW,), jnp.float32),
            pltpu.VMEM((N_PAD,), jnp.float32),
            pltpu.VMEM((RPB, CW), jnp.int32),
            pltpu.VMEM((RPB, CW), jnp.int32),
            pltpu.VMEM((BLKE,), jnp.float32),
            pltpu.VMEM((LANES,), jnp.float32),
            pltpu.VMEM((NS, LANES), jnp.float32),
            pltpu.VMEM_SHARED((NS, LANES), jnp.float32),
        ],
        compiler_params=_sc_compiler_params(),
    )
    return kern(s_pad, d_pad, src3, dst3)


def _scb_body(z_hbm, src_hbm, dst_hbm, ex_hbm, zeros_hbm,
              p_hbm,
              srcblk_v, dstblk_v, eexp_blk,
              z0, z1, z2, z3,
              p_sh,
              sg0, sg1, sg2, sg3, ss0, ss1, ss2, ss3):
    cid = lax.axis_index("c")
    sid = lax.axis_index("s")
    wid = cid * NS + sid
    cbase = sid * ROWS_W

    pltpu.sync_copy(zeros_hbm, p_sh.at[pl.ds(cbase, ROWS_W)])
    plsc.subcore_barrier()

    zbufs = (z0, z1, z2, z3)
    gsems = (sg0, sg1, sg2, sg3)
    ssems = (ss0, ss1, ss2, ss3)
    NBUF = 4

    def scale_chunk(buf, r):
        @plsc.parallel_loop(0, CW, unroll=4)
        def _(j):
            bc = plsc.load_gather(
                eexp_blk, [jnp.zeros((LANES,), jnp.int32) + (r * CW + j)])
            row = buf.at[j]
            for t in range(D // LANES):
                row[pl.ds(t * LANES, LANES)] = (
                    row[pl.ds(t * LANES, LANES)] * bc)

    @pl.loop(0, NBLK)
    def _(b):
        pltpu.sync_copy(src_hbm.at[wid, pl.ds(b * RPB, RPB)], srcblk_v)
        pltpu.sync_copy(dst_hbm.at[wid, pl.ds(b * RPB, RPB)], dstblk_v)
        pltpu.sync_copy(ex_hbm.at[pl.ds(wid * EPW + b * BLKE, BLKE)], eexp_blk)

        gds = [None] * RPB
        sds = [None] * RPB
        for r in range(NBUF - 1):
            gds[r] = pltpu.async_copy(
                z_hbm.at[srcblk_v.at[r]], zbufs[r % NBUF], gsems[r % NBUF])
        for r in range(RPB):
            buf = zbufs[r % NBUF]
            gds[r].wait()
            scale_chunk(buf, r)
            sds[r] = pltpu.async_copy(
                buf, p_sh.at[dstblk_v.at[r]], ssems[r % NBUF], add=True)
            if r + NBUF - 1 < RPB:
                nb = (r + NBUF - 1) % NBUF
                if r >= 1 and sds[r - 1] is not None:
                    sds[r - 1].wait()
                gds[r + NBUF - 1] = pltpu.async_copy(
                    z_hbm.at[srcblk_v.at[r + NBUF - 1]], zbufs[nb], gsems[nb])
        for r in range(RPB - NBUF, RPB):
            if r >= 0 and sds[r] is not None:
                sds[r].wait()

    plsc.subcore_barrier()

    pltpu.sync_copy(p_sh.at[pl.ds(cbase, ROWS_W)],
                    p_hbm.at[cid, pl.ds(cbase, ROWS_W)])


def _sc_aggregate(z_pad, src3, dst3, ex_all, zeros_rows):
    mesh = plsc.VectorSubcoreMesh(core_axis_name="c", subcore_axis_name="s")
    kern = pl.kernel(
        _scb_body,
        out_type=[
            jax.ShapeDtypeStruct((NC, N_PAD, D), jnp.float32),
        ],
        mesh=mesh,
        scratch_types=[
            pltpu.VMEM((RPB, CW), jnp.int32),
            pltpu.VMEM((RPB, CW), jnp.int32),
            pltpu.VMEM((BLKE,), jnp.float32),
            pltpu.VMEM((CW, D), jnp.float32),
            pltpu.VMEM((CW, D), jnp.float32),
            pltpu.VMEM((CW, D), jnp.float32),
            pltpu.VMEM((CW, D), jnp.float32),
            pltpu.VMEM_SHARED((N_PAD, D), jnp.float32),
            pltpu.SemaphoreType.DMA,
            pltpu.SemaphoreType.DMA,
            pltpu.SemaphoreType.DMA,
            pltpu.SemaphoreType.DMA,
            pltpu.SemaphoreType.DMA,
            pltpu.SemaphoreType.DMA,
            pltpu.SemaphoreType.DMA,
            pltpu.SemaphoreType.DMA,
        ],
        compiler_params=_sc_compiler_params(),
    )
    return kern(z_pad, src3, dst3, ex_all, zeros_rows)


def _combine_body(p0_ref, p1_ref, d_ref, c_ref, o_ref):
    c0 = c_ref[0]
    c1 = c_ref[1]
    den = (jnp.sum(d_ref[0], axis=0) * c0
           + jnp.sum(d_ref[1], axis=0) * c1)[:, None]
    den = jnp.where(den == 0.0, jnp.float32(1.0), den)
    o_ref[...] = (p0_ref[...] * c0 + p1_ref[...] * c1) / den


def _combine(p0, p1, dall, c):
    blk = 1024
    return pl.pallas_call(
        _combine_body,
        grid=(N_PAD // blk,),
        in_specs=[
            pl.BlockSpec((blk, D), lambda i: (i, 0)),
            pl.BlockSpec((blk, D), lambda i: (i, 0)),
            pl.BlockSpec((NC, NS, blk), lambda i: (0, 0, i)),
            pl.BlockSpec(memory_space=pltpu.SMEM),
        ],
        out_specs=pl.BlockSpec((blk, D), lambda i: (i, 0)),
        out_shape=jax.ShapeDtypeStruct((N_PAD, D), jnp.float32),
    )(p0, p1, dall, c)


@jax.jit
def kernel(h, edge_index, W_fc, W_attn):
    a_src = W_attn[0, :D]
    a_dst = W_attn[0, D:]
    wt = W_fc.T
    v1 = wt @ a_src
    v2 = wt @ a_dst
    w2 = jnp.zeros((D, 256), jnp.float32)
    w2 = w2.at[:, :D].set(wt).at[:, D].set(v1).at[:, D + 1].set(v2)

    z2 = _project(h, w2)
    z_pad = jnp.zeros((N_PAD, D), jnp.float32).at[:N_NODES].set(z2[:, :D])
    s_pad = (jnp.zeros((N_PAD,), jnp.float32).at[:N_NODES].set(z2[:, D])
             .at[SENT].set(jnp.float32(SENT_VAL)))
    d_pad = (jnp.zeros((N_PAD,), jnp.float32).at[:N_NODES].set(z2[:, D + 1])
             .at[SENT].set(jnp.float32(SENT_VAL)))

    def slab(row):
        a = row.astype(jnp.int32).reshape(NW, N_EDGES // NW)
        a = jnp.pad(a, ((0, 0), (0, EPW - N_EDGES // NW)),
                    constant_values=SENT)
        return a.reshape(NW, EPW // CW, CW)

    src3 = slab(edge_index[0])
    dst3 = slab(edge_index[1])
    zeros_rows = jnp.zeros((ROWS_W, D), jnp.float32)

    ex_all, den, mx = _sc_logits(s_pad, d_pad, src3, dst3)
    (p,) = _sc_aggregate(z_pad, src3, dst3, ex_all, zeros_rows)
    dall = den.reshape(NC, NS, N_PAD)

    m0 = jnp.max(mx[0, 0])
    m1 = jnp.max(mx[1, 0])
    mg = jnp.maximum(m0, m1)
    c = jnp.stack([jnp.exp(m0 - mg), jnp.exp(m1 - mg)])

    sh = _combine(p[0], p[1], dall, c)
    return sh[:N_NODES]
